# Initial kernel scaffold; baseline (speedup 1.0000x reference)
import functools

import jax
import jax.numpy as jnp
from jax import lax
from jax.experimental import pallas as pl
from jax.experimental.pallas import tpu as pltpu
from jax.experimental.pallas import tpu_sc as plsc

_GRP = 128
_GPC = 4
_CHUNK = _GRP * _GPC


def _gather_kernel(B, V, D, n_chunks, b_per_w, NC):
    mesh = plsc.VectorSubcoreMesh(core_axis_name="c", subcore_axis_name="s")

    @functools.partial(
        pl.kernel,
        mesh=mesh,
        out_type=jax.ShapeDtypeStruct((B, D), jnp.float32),
        scratch_types=[
            pltpu.VMEM((2, _GPC, _GRP), jnp.int32),
            pltpu.VMEM((2, _CHUNK, D), jnp.float32),
            pltpu.SemaphoreType.DMA,
            pltpu.SemaphoreType.DMA,
        ],
    )
    def k(x_hbm, tab_hbm, out_hbm, idx_v, rows_v, gsem, osem):
        wid = lax.axis_index("s") * NC + lax.axis_index("c")
        base = wid * b_per_w

        def fire_chunk(ci, slot):
            row0 = (base + ci * _CHUNK) // _GRP
            pltpu.sync_copy(x_hbm.at[pl.ds(row0, _GPC)], idx_v.at[slot])
            for j in range(_GPC):
                pltpu.async_copy(
                    tab_hbm.at[idx_v.at[slot].at[j]],
                    rows_v.at[slot].at[pl.ds(j * _GRP, _GRP)],
                    gsem,
                )

        def drain_gathers(slot):
            for j in range(_GPC):
                pltpu.make_async_copy(
                    tab_hbm.at[idx_v.at[slot].at[j]],
                    rows_v.at[slot].at[pl.ds(j * _GRP, _GRP)],
                    gsem,
                ).wait()

        def writeback(ci, slot):
            off = base + ci * _CHUNK
            pltpu.async_copy(rows_v.at[slot], out_hbm.at[pl.ds(off, _CHUNK)], osem)

        def drain_writeback(ci, slot):
            off = base + ci * _CHUNK
            pltpu.make_async_copy(
                rows_v.at[slot], out_hbm.at[pl.ds(off, _CHUNK)], osem
            ).wait()

        fire_chunk(0, 0)

        def body(i, _):
            slot = lax.rem(i, 2)
            nxt = 1 - slot

            @pl.when(i + 1 < n_chunks)
            def _():
                fire_chunk(i + 1, nxt)

            drain_gathers(slot)

            @pl.when(i >= 2)
            def _():
                drain_writeback(i - 2, slot)

            writeback(i, slot)
            return 0

        lax.fori_loop(0, n_chunks, body, 0, unroll=False)

        @pl.when(n_chunks >= 2)
        def _():
            drain_writeback(n_chunks - 2, lax.rem(n_chunks - 2, 2))

        drain_writeback(n_chunks - 1, lax.rem(n_chunks - 1, 2))

    return k


def kernel(x, table):
    B0, S = x.shape
    V, D = table.shape
    B = B0 * S

    info = plsc.get_sparse_core_info()
    NC, NS = info.num_cores, info.num_subcores
    NW = NC * NS
    assert B % (NW * _CHUNK) == 0
    b_per_w = B // NW
    n_chunks = b_per_w // _CHUNK

    xf = x.reshape(B // _GRP, _GRP).astype(jnp.int32)
    k = _gather_kernel(B, V, D, n_chunks, b_per_w, NC)
    out = k(xf, table)
    return out.reshape(B0, S, D)


# speedup vs baseline: 1.8741x; 1.8741x over previous
import functools

import jax
import jax.numpy as jnp
from jax import lax
from jax.experimental import pallas as pl
from jax.experimental.pallas import tpu as pltpu
from jax.experimental.pallas import tpu_sc as plsc

_GRP = 128
_GPC = 4
_CHUNK = _GRP * _GPC
_PAIR = 2 * _CHUNK


def _gather_kernel(B, V, D, n_chunks, b_per_w, NC):
    mesh = plsc.VectorSubcoreMesh(core_axis_name="c", subcore_axis_name="s")
    n_pairs = n_chunks // 2
    assert n_chunks % 2 == 0 and n_pairs >= 2

    @functools.partial(
        pl.kernel,
        mesh=mesh,
        compiler_params=pltpu.CompilerParams(use_tc_tiling_on_sc=False),
        out_type=jax.ShapeDtypeStruct((B, D), jnp.float32),
        scratch_types=[
            pltpu.VMEM((2 * _GPC, _GRP), jnp.int32),
            pltpu.VMEM((2, _CHUNK, D), jnp.float32),
            pltpu.SemaphoreType.DMA,
            pltpu.SemaphoreType.DMA,
        ],
    )
    def k(x_hbm, tab_hbm, out_hbm, idx_v, rows_v, gsem, osem):
        wid = lax.axis_index("s") * NC + lax.axis_index("c")
        base = wid * b_per_w

        def stage_pair(h):
            row0 = pl.multiple_of((base + h * _PAIR) // _GRP, 8)
            pltpu.sync_copy(x_hbm.at[pl.ds(row0, 2 * _GPC)], idx_v)

        def fire_gathers(half, rslot):
            for j in range(_GPC):
                pltpu.async_copy(
                    tab_hbm.at[idx_v.at[half * _GPC + j]],
                    rows_v.at[rslot].at[pl.ds(j * _GRP, _GRP)],
                    gsem,
                )

        def drain_gathers(half, rslot):
            for j in range(_GPC):
                pltpu.make_async_copy(
                    tab_hbm.at[idx_v.at[half * _GPC + j]],
                    rows_v.at[rslot].at[pl.ds(j * _GRP, _GRP)],
                    gsem,
                ).wait()

        def writeback(ci, rslot):
            off = base + ci * _CHUNK
            pltpu.async_copy(rows_v.at[rslot], out_hbm.at[pl.ds(off, _CHUNK)], osem)

        def drain_writeback(ci, rslot):
            off = base + ci * _CHUNK
            pltpu.make_async_copy(
                rows_v.at[rslot], out_hbm.at[pl.ds(off, _CHUNK)], osem
            ).wait()

        stage_pair(0)
        fire_gathers(0, 0)

        def body(h, _):
            a = h * 2

            @pl.when(a >= 1)
            def _():
                drain_writeback(a - 1, 1)

            fire_gathers(1, 1)
            drain_gathers(0, 0)
            writeback(a, 0)

            drain_writeback(a, 0)
            drain_gathers(1, 1)
            writeback(a + 1, 1)

            @pl.when(h + 1 < n_pairs)
            def _():
                stage_pair(h + 1)
                fire_gathers(0, 0)
            return 0

        lax.fori_loop(0, n_pairs, body, 0)
        drain_writeback(n_chunks - 1, 1)

    return k


def kernel(x, table):
    B0, S = x.shape
    V, D = table.shape
    B = B0 * S

    info = plsc.get_sparse_core_info()
    NC, NS = info.num_cores, info.num_subcores
    NW = NC * NS
    assert B % (NW * _PAIR) == 0
    b_per_w = B // NW
    n_chunks = b_per_w // _CHUNK

    xf = x.reshape(B // _GRP, _GRP).astype(jnp.int32)
    k = _gather_kernel(B, V, D, n_chunks, b_per_w, NC)
    out = k(xf, table)
    return out.reshape(B0, S, D)
